# baseline (speedup 1.0000x reference)
import functools

import jax
import jax.numpy as jnp
from jax import lax
from jax.experimental import pallas as pl
from jax.experimental.pallas import tpu as pltpu
from jax.experimental.pallas import tpu_sc as plsc

_GAMMA = 0.9
_N = 16384
_G = 2
_NG = _N // _G
_C = 1000
_W = 512
_CHUNK = 64
_NCHUNK = _NG // (16 * _CHUNK)
_UNITS = _C // 8


def _zero16():
    return jnp.zeros((16,), jnp.float32)


def _body(preds_hbm, labels_hbm, sums_hbm, cnt_hbm,
          buf2, idx2, ones_v, z8, sems, acc_sp, cnt_sp):
    cid = lax.axis_index("c")
    sid = lax.axis_index("s")
    col0 = cid * (_C - _W)

    base = sid * (_NG // 16)

    def _start_fetch(k, slot):
        r0 = base + k * _CHUNK
        pltpu.async_copy(labels_hbm.at[pl.ds(r0, _CHUNK)], idx2.at[slot],
                         sems.at[slot])
        pltpu.async_copy(preds_hbm.at[pl.ds(r0, _CHUNK), pl.ds(col0, _W)],
                         buf2.at[slot], sems.at[slot])

    def _wait_fetch(slot):
        pltpu.make_async_copy(labels_hbm.at[pl.ds(0, _CHUNK)], idx2.at[slot],
                              sems.at[slot]).wait()
        pltpu.make_async_copy(preds_hbm.at[pl.ds(0, _CHUNK), pl.ds(0, _W)],
                              buf2.at[slot], sems.at[slot]).wait()

    _start_fetch(0, 0)
    _start_fetch(1, 1)

    def _zrow(r, _):
        def _zc(c, _):
            z8[r, pl.ds(c * 16, 16)] = _zero16()
            return 0
        return lax.fori_loop(0, _W // 16, _zc, 0)
    lax.fori_loop(0, 8, _zrow, 0)

    def _obuf(i, _):
        ones_v[pl.ds(i * 16, 16)] = _zero16() + 1.0
        return 0
    lax.fori_loop(0, _CHUNK // 16, _obuf, 0)

    def _zunit(j, _):
        u = sid + j * 16

        @pl.when(u < _UNITS)
        def _():
            pltpu.sync_copy(z8, acc_sp.at[pl.ds(u * 8, 8)])
        return 0
    lax.fori_loop(0, (_UNITS + 15) // 16, _zunit, 0)

    @pl.when(sid < 8)
    def _():
        pltpu.sync_copy(z8.at[0, pl.ds(0, 128)],
                        cnt_sp.at[pl.ds(sid * 128, 128)])

    plsc.subcore_barrier()

    for k in range(_NCHUNK):
        slot = k % 2
        _wait_fetch(slot)
        pltpu.sync_copy(buf2.at[slot], acc_sp.at[idx2.at[slot]], add=True)
        pltpu.sync_copy(ones_v, cnt_sp.at[idx2.at[slot]], add=True)
        if k + 2 < _NCHUNK:
            _start_fetch(k + 2, slot)

    plsc.subcore_barrier()

    def _wunit(j, _):
        u = sid + j * 16

        @pl.when(u < _UNITS)
        def _():
            pltpu.sync_copy(acc_sp.at[pl.ds(u * 8, 8)],
                            sums_hbm.at[pl.ds(u * 8, 8), pl.ds(col0, _W)])
        return 0
    lax.fori_loop(0, (_UNITS + 15) // 16, _wunit, 0)

    @pl.when(jnp.logical_and(cid == 0, sid < 8))
    def _():
        pltpu.sync_copy(cnt_sp.at[pl.ds(sid * 128, 128)],
                        cnt_hbm.at[pl.ds(sid * 128, 128)])


@functools.partial(
    pl.kernel,
    out_type=(
        jax.ShapeDtypeStruct((_C, _C), jnp.float32),
        jax.ShapeDtypeStruct((1024,), jnp.float32),
    ),
    mesh=plsc.VectorSubcoreMesh(core_axis_name="c", subcore_axis_name="s"),
    compiler_params=pltpu.CompilerParams(
        use_tc_tiling_on_sc=False, needs_layout_passes=False),
    scratch_types=[
        pltpu.VMEM((2, _CHUNK, _W), jnp.float32),
        pltpu.VMEM((2, _CHUNK), jnp.int32),
        pltpu.VMEM((_CHUNK,), jnp.float32),
        pltpu.VMEM((8, _W), jnp.float32),
        pltpu.SemaphoreType.DMA((2,)),
        pltpu.VMEM_SHARED((_C, _W), jnp.float32),
        pltpu.VMEM_SHARED((1024,), jnp.float32),
    ],
)
def _segment_sums(preds_hbm, labels_hbm, sums_hbm, cnt_hbm, *scratch):
    _body(preds_hbm, labels_hbm, sums_hbm, cnt_hbm, *scratch)


_BLK = 200


def _ema_body(s0_ref, s1_ref, c0_ref, c1_ref, vp_ref, out_ref):
    s = s0_ref[...] + s1_ref[...]
    cnt = c0_ref[...] + c1_ref[...]
    has = cnt > 0.0
    a = jnp.where(has, (1.0 - _GAMMA) / jnp.where(has, cnt, 1.0), 0.0)
    b = jnp.where(has, jnp.float32(_GAMMA), 1.0)
    out_ref[...] = s * a + vp_ref[...] * b


_ema = pl.pallas_call(
    _ema_body,
    grid=(_C // _BLK,),
    in_specs=[
        pl.BlockSpec((_BLK, _C), lambda i: (i, 0)),
        pl.BlockSpec((_BLK, _C), lambda i: (i, 0)),
        pl.BlockSpec((_BLK, 1), lambda i: (i, 0)),
        pl.BlockSpec((_BLK, 1), lambda i: (i, 0)),
        pl.BlockSpec((_BLK, _C), lambda i: (i, 0)),
    ],
    out_specs=pl.BlockSpec((_BLK, _C), lambda i: (i, 0)),
    out_shape=jax.ShapeDtypeStruct((_C, _C), jnp.float32),
)


def kernel(preds, labels, val_preds):
    s0, c0 = _segment_sums(preds[:_NG], labels[:_NG])
    s1, c1 = _segment_sums(preds[_NG:], labels[_NG:])
    return _ema(s0, s1, c0[:_C, None], c1[:_C, None], val_preds)


# speedup vs baseline: 1.2491x; 1.2491x over previous
import functools

import jax
import jax.numpy as jnp
from jax import lax
from jax.experimental import pallas as pl
from jax.experimental.pallas import tpu as pltpu
from jax.experimental.pallas import tpu_sc as plsc

_GAMMA = 0.9
_N = 16384
_C = 1000
_W = 512
_CHUNK = 64
_NSLOT = 2
_NCHUNK = _N // (16 * _CHUNK)
_UNITS = _C // 8


def _zero16():
    return jnp.zeros((16,), jnp.float32)


def _body(preds_hbm, labels_hbm, sums_hbm, cnt_hbm,
          buf2, idx2, ones_v, z8, sems, acc_sp, cnt_sp):
    cid = lax.axis_index("c")
    sid = lax.axis_index("s")
    col0 = cid * (_C - _W)

    base = sid * (_N // 16)

    def _start_fetch(k, slot):
        r0 = base + k * _CHUNK
        pltpu.async_copy(labels_hbm.at[pl.ds(r0, _CHUNK)], idx2.at[slot],
                         sems.at[slot])
        pltpu.async_copy(preds_hbm.at[pl.ds(r0, _CHUNK), pl.ds(col0, _W)],
                         buf2.at[slot], sems.at[slot])

    def _wait_fetch(slot):
        pltpu.make_async_copy(labels_hbm.at[pl.ds(0, _CHUNK)], idx2.at[slot],
                              sems.at[slot]).wait()
        pltpu.make_async_copy(preds_hbm.at[pl.ds(0, _CHUNK), pl.ds(0, _W)],
                              buf2.at[slot], sems.at[slot]).wait()

    for k in range(_NSLOT):
        _start_fetch(k, k)

    def _zrow(r, _):
        def _zc(c, _):
            z8[r, pl.ds(c * 16, 16)] = _zero16()
            return 0
        return lax.fori_loop(0, _W // 16, _zc, 0)
    lax.fori_loop(0, 8, _zrow, 0)

    def _obuf(i, _):
        ones_v[pl.ds(i * 16, 16)] = _zero16() + 1.0
        return 0
    lax.fori_loop(0, _CHUNK // 16, _obuf, 0)

    def _zunit(j, _):
        u = sid + j * 16

        @pl.when(u < _UNITS)
        def _():
            pltpu.sync_copy(z8, acc_sp.at[pl.ds(u * 8, 8)])
        return 0
    lax.fori_loop(0, (_UNITS + 15) // 16, _zunit, 0)

    @pl.when(sid < 8)
    def _():
        pltpu.sync_copy(z8.at[0, pl.ds(0, 128)],
                        cnt_sp.at[pl.ds(sid * 128, 128)])

    plsc.subcore_barrier()

    is0 = cid == 0
    for k in range(_NCHUNK):
        slot = k % _NSLOT
        _wait_fetch(slot)
        pltpu.sync_copy(buf2.at[slot], acc_sp.at[idx2.at[slot]], add=True)

        @pl.when(is0)
        def _():
            pltpu.sync_copy(ones_v, cnt_sp.at[idx2.at[slot]], add=True)
        if k + _NSLOT < _NCHUNK:
            _start_fetch(k + _NSLOT, slot)

    plsc.subcore_barrier()

    def _wunit(j, _):
        u = sid + j * 16

        @pl.when(u < _UNITS)
        def _():
            pltpu.sync_copy(acc_sp.at[pl.ds(u * 8, 8)],
                            sums_hbm.at[pl.ds(u * 8, 8), pl.ds(col0, _W)])
        return 0
    lax.fori_loop(0, (_UNITS + 15) // 16, _wunit, 0)

    @pl.when(jnp.logical_and(cid == 0, sid < 8))
    def _():
        pltpu.sync_copy(cnt_sp.at[pl.ds(sid * 128, 128)],
                        cnt_hbm.at[pl.ds(sid * 128, 128)])


@functools.partial(
    pl.kernel,
    out_type=(
        jax.ShapeDtypeStruct((_C, _C), jnp.float32),
        jax.ShapeDtypeStruct((1024,), jnp.float32),
    ),
    mesh=plsc.VectorSubcoreMesh(core_axis_name="c", subcore_axis_name="s"),
    compiler_params=pltpu.CompilerParams(
        use_tc_tiling_on_sc=False, needs_layout_passes=False),
    scratch_types=[
        pltpu.VMEM((_NSLOT, _CHUNK, _W), jnp.float32),
        pltpu.VMEM((_NSLOT, _CHUNK), jnp.int32),
        pltpu.VMEM((_CHUNK,), jnp.float32),
        pltpu.VMEM((8, _W), jnp.float32),
        pltpu.SemaphoreType.DMA((_NSLOT,)),
        pltpu.VMEM_SHARED((_C, _W), jnp.float32),
        pltpu.VMEM_SHARED((1024,), jnp.float32),
    ],
)
def _segment_sums(preds_hbm, labels_hbm, sums_hbm, cnt_hbm, *scratch):
    _body(preds_hbm, labels_hbm, sums_hbm, cnt_hbm, *scratch)


_BLK = 200


def _ema_body(s_ref, c_ref, vp_ref, out_ref):
    s = s_ref[...]
    cnt = c_ref[...]
    has = cnt > 0.0
    a = jnp.where(has, (1.0 - _GAMMA) / jnp.where(has, cnt, 1.0), 0.0)
    b = jnp.where(has, jnp.float32(_GAMMA), 1.0)
    out_ref[...] = s * a + vp_ref[...] * b


_ema = pl.pallas_call(
    _ema_body,
    grid=(_C // _BLK,),
    in_specs=[
        pl.BlockSpec((_BLK, _C), lambda i: (i, 0)),
        pl.BlockSpec((_BLK, 1), lambda i: (i, 0)),
        pl.BlockSpec((_BLK, _C), lambda i: (i, 0)),
    ],
    out_specs=pl.BlockSpec((_BLK, _C), lambda i: (i, 0)),
    out_shape=jax.ShapeDtypeStruct((_C, _C), jnp.float32),
)


def kernel(preds, labels, val_preds):
    sums, cnt = _segment_sums(preds, labels)
    return _ema(sums, cnt[:_C, None], val_preds)
